# Initial kernel scaffold; baseline (speedup 1.0000x reference)
import jax
import jax.numpy as jnp
from jax.experimental import pallas as pl


def kernel(ctrl_pts, knot_u):
    raise NotImplementedError("write your pallas kernel here")


# speedup vs baseline: 76.4710x; 76.4710x over previous
import functools

import jax
import jax.numpy as jnp
from jax import lax
from jax.experimental import pallas as pl
from jax.experimental.pallas import tpu as pltpu, tpu_sc as plsc

P = 3
OUT_DIM = 4096
N_CTRL = 8192
N_KNOTS = N_CTRL + P + 1
N_UP = N_KNOTS - P
KNOTS_PAD = 8208

NC, NS, L = 2, 16, 16
NW = NC * NS
S_PER_W = OUT_DIM // NW
V_PER_W = S_PER_W // L

U_START = 1e-5
U_STEP = (1.0 - 2e-5) / (OUT_DIM - 1)
EPS = 1e-8
DEG_EPS = 1e-4


def _basis_step(Nr, U1, U2, u, saved):
    dU = (U1 - u) + (u - U2)
    zero = dU == 0.0
    dU_ = jnp.where(zero, DEG_EPS, dU)
    temp = Nr / dU_
    temp = jnp.where(zero, DEG_EPS, temp)
    return saved + (U1 - u) * temp, (u - U2) * temp


def _sc_body(ctrl_hbm, knots_hbm, out_hbm, knots_v, ctrl_v, outbuf):
    wid = lax.axis_index("s") * NC + lax.axis_index("c")
    pltpu.sync_copy(knots_hbm, knots_v)
    pltpu.sync_copy(ctrl_hbm, ctrl_v)

    lanes = lax.iota(jnp.int32, L)

    def gk(idx):
        safe = jnp.minimum(jnp.maximum(idx, 0), N_KNOTS - 1)
        return plsc.load_gather(knots_v, [safe])

    for v in range(V_PER_W):
        i = wid * S_PER_W + (v * L) + lanes
        u = U_START + i.astype(jnp.float32) * U_STEP

        m = jnp.zeros((L,), jnp.int32)
        step = 8192
        while step >= 1:
            cand = m + step
            valid = cand <= N_UP
            j = jnp.minimum(cand, N_UP) - 1
            vvals = gk(j + P)
            pred = valid & ((u - vvals) > EPS)
            m = jnp.where(pred, cand, m)
            step //= 2

        d = u - gk(jnp.maximum(m, 1) + P - 1)
        off = jnp.where(
            m == 0,
            jnp.zeros((L,), jnp.int32),
            jnp.where((d > 1.0) & (m < N_UP), m, m - 1),
        )
        uspan = off + P

        A1, A2, A3 = gk(uspan + 1), gk(uspan + 2), gk(uspan + 3)
        B0, B1, B2 = gk(uspan), gk(uspan - 1), gk(uspan - 2)

        zero = jnp.zeros((L,), jnp.float32)
        N0 = jnp.ones((L,), jnp.float32)
        N0, s = _basis_step(N0, A1, B0, u, zero)
        N1 = s
        N0, s = _basis_step(N0, A1, B1, u, zero)
        N1, s = _basis_step(N1, A2, B0, u, s)
        N2 = s
        N0, s = _basis_step(N0, A1, B2, u, zero)
        N1, s = _basis_step(N1, A2, B1, u, s)
        N2, s = _basis_step(N2, A3, B0, u, s)
        N3 = s

        r0 = jnp.minimum(off, N_CTRL - 1)
        r1 = jnp.minimum(off + 1, N_CTRL - 1)
        r2 = jnp.minimum(off + 2, N_CTRL - 1)
        r3 = jnp.minimum(off + 3, N_CTRL - 1)
        base_out = (v * L + lanes) * 3
        for dim in range(3):
            c0 = plsc.load_gather(ctrl_v, [r0 * 3 + dim])
            c1 = plsc.load_gather(ctrl_v, [r1 * 3 + dim])
            c2 = plsc.load_gather(ctrl_v, [r2 * 3 + dim])
            c3 = plsc.load_gather(ctrl_v, [r3 * 3 + dim])
            val = ((N0 * c0 + N1 * c1) + N2 * c2) + N3 * c3
            plsc.store_scatter(outbuf, [base_out + dim], val)

    pltpu.sync_copy(outbuf, out_hbm.at[pl.ds(wid * S_PER_W * 3, S_PER_W * 3)])


@jax.jit
def _launch(ctrl_flat, knots_pad):
    mesh = plsc.VectorSubcoreMesh(core_axis_name="c", subcore_axis_name="s")
    run = functools.partial(
        pl.kernel,
        mesh=mesh,
        out_type=jax.ShapeDtypeStruct((OUT_DIM * 3,), jnp.float32),
        scratch_types=[
            pltpu.VMEM((KNOTS_PAD,), jnp.float32),
            pltpu.VMEM((N_CTRL * 3,), jnp.float32),
            pltpu.VMEM((S_PER_W * 3,), jnp.float32),
        ],
        compiler_params=pltpu.CompilerParams(needs_layout_passes=False),
    )(_sc_body)
    return run(ctrl_flat, knots_pad)


def kernel(ctrl_pts, knot_u):
    ctrl_flat = ctrl_pts.reshape(N_CTRL * 3)
    knots_pad = jnp.pad(knot_u.reshape(N_KNOTS), (0, KNOTS_PAD - N_KNOTS))
    out = _launch(ctrl_flat, knots_pad)
    return out.reshape(1, OUT_DIM, 3)
